# Initial kernel scaffold; baseline (speedup 1.0000x reference)
import jax
import jax.numpy as jnp
from jax.experimental import pallas as pl


def kernel(ContTensor, CatTensor, LabelTensor, DoseTensor, TimeDiffTensor, VTensor, VancoElTensor, PtList, LengList, embed_weight):
    raise NotImplementedError("write your pallas kernel here")


# speedup vs baseline: 1.2301x; 1.2301x over previous
import functools

import jax
import jax.numpy as jnp
from jax import lax
from jax.experimental import pallas as pl
from jax.experimental.pallas import tpu as pltpu
from jax.experimental.pallas import tpu_sc as plsc

B, T, NC, DC = 1024, 50, 26, 16
V, D = 1000000, 64
ROWS = B * T
DOUT = D + DC
L = 16

_NUM_CORES = 2
_NUM_SUBCORES = 16
NW = _NUM_CORES * _NUM_SUBCORES
ROWS_PER_W = ROWS // NW
CR = 4
IDXC = CR * NC
NCHUNK = ROWS_PER_W // CR

_mesh = plsc.VectorSubcoreMesh(core_axis_name="c", subcore_axis_name="s")


@functools.partial(
    pl.kernel,
    mesh=_mesh,
    out_type=jax.ShapeDtypeStruct((ROWS, DOUT), jnp.float32),
    compiler_params=pltpu.CompilerParams(use_tc_tiling_on_sc=False),
    scratch_types=[
        pltpu.VMEM((IDXC,), jnp.int32),
        pltpu.VMEM((IDXC, D), jnp.float32),
        pltpu.VMEM((CR, DC), jnp.float32),
        pltpu.VMEM((CR, DOUT), jnp.float32),
        pltpu.SemaphoreType.DMA,
    ],
)
def _emb_kernel(idx_hbm, cont_hbm, table_hbm, out_hbm,
                idx_v, rows_v, cont_v, out_v, sem):
    wid = lax.axis_index("s") * _NUM_CORES + lax.axis_index("c")
    base = wid * ROWS_PER_W

    def body(g, carry):
        row0 = base + g * CR
        pltpu.sync_copy(idx_hbm.at[pl.ds(row0 * NC, IDXC)], idx_v)
        pltpu.async_copy(table_hbm.at[idx_v], rows_v, sem).wait()
        pltpu.sync_copy(cont_hbm.at[pl.ds(row0, CR)], cont_v)
        for r in range(CR):
            for k in range(D // L):
                acc = rows_v[r * NC, pl.ds(k * L, L)]
                for j in range(1, NC):
                    acc = acc + rows_v[r * NC + j, pl.ds(k * L, L)]
                out_v[r, pl.ds(k * L, L)] = acc
            out_v[r, pl.ds(D, DC)] = cont_v[r, :]
        pltpu.sync_copy(out_v, out_hbm.at[pl.ds(row0, CR)])
        return carry

    lax.fori_loop(0, NCHUNK, body, 0)


def kernel(ContTensor, CatTensor, LabelTensor, DoseTensor, TimeDiffTensor,
           VTensor, VancoElTensor, PtList, LengList, embed_weight):
    idx_flat = CatTensor.astype(jnp.int32).reshape(ROWS * NC)
    cont_flat = ContTensor.reshape(ROWS, DC)
    out = _emb_kernel(idx_flat, cont_flat, embed_weight)
    outEmb = out.reshape(B, T, DOUT)
    return (outEmb, LabelTensor, LengList, DoseTensor, TimeDiffTensor,
            VTensor, VancoElTensor, PtList)


# speedup vs baseline: 1.7975x; 1.4613x over previous
import functools

import jax
import jax.numpy as jnp
from jax import lax
from jax.experimental import pallas as pl
from jax.experimental.pallas import tpu as pltpu
from jax.experimental.pallas import tpu_sc as plsc

B, T, NC, DC = 1024, 50, 26, 16
V, D = 1000000, 64
ROWS = B * T
DOUT = D + DC
L = 16

_NUM_CORES = 2
_NUM_SUBCORES = 16
NW = _NUM_CORES * _NUM_SUBCORES
ROWS_PER_W = ROWS // NW
CR = 4
IDXC = CR * NC
NCHUNK = ROWS_PER_W // CR
NBUF = 2

_mesh = plsc.VectorSubcoreMesh(core_axis_name="c", subcore_axis_name="s")


@functools.partial(
    pl.kernel,
    mesh=_mesh,
    out_type=jax.ShapeDtypeStruct((ROWS, DOUT), jnp.float32),
    compiler_params=pltpu.CompilerParams(use_tc_tiling_on_sc=False),
    scratch_types=[
        pltpu.VMEM((NBUF, IDXC), jnp.int32),
        pltpu.VMEM((NBUF, IDXC, D), jnp.float32),
        pltpu.VMEM((ROWS_PER_W, DC), jnp.float32),
        pltpu.VMEM((NBUF, CR, DOUT), jnp.float32),
        pltpu.SemaphoreType.DMA((NBUF,)),
        pltpu.SemaphoreType.DMA((NBUF,)),
        pltpu.SemaphoreType.DMA((NBUF,)),
    ],
)
def _emb_kernel(idx_hbm, cont_hbm, table_hbm, out_hbm,
                idx_v, rows_v, cont_v, out_v, idx_sem, gather_sem, out_sem):
    wid = lax.axis_index("s") * _NUM_CORES + lax.axis_index("c")
    base = wid * ROWS_PER_W

    def issue_idx(c, b):
        pltpu.async_copy(idx_hbm.at[pl.ds((base + c * CR) * NC, IDXC)],
                         idx_v.at[b], idx_sem.at[b])

    def wait_idx(c, b):
        pltpu.make_async_copy(idx_hbm.at[pl.ds((base + c * CR) * NC, IDXC)],
                              idx_v.at[b], idx_sem.at[b]).wait()

    def issue_gather(b):
        pltpu.async_copy(table_hbm.at[idx_v.at[b]], rows_v.at[b],
                         gather_sem.at[b])

    def wait_gather(b):
        pltpu.make_async_copy(table_hbm.at[idx_v.at[b]], rows_v.at[b],
                              gather_sem.at[b]).wait()

    def issue_out(c, b):
        pltpu.async_copy(out_v.at[b], out_hbm.at[pl.ds(base + c * CR, CR)],
                         out_sem.at[b])

    def wait_out(c, b):
        pltpu.make_async_copy(out_v.at[b], out_hbm.at[pl.ds(base + c * CR, CR)],
                              out_sem.at[b]).wait()

    pltpu.sync_copy(cont_hbm.at[pl.ds(base, ROWS_PER_W)], cont_v)
    for b in range(NBUF):
        issue_idx(b, b)
    wait_idx(0, 0)
    issue_gather(0)

    def body(ii, carry):
        for b in range(NBUF):
            c = ii * NBUF + b
            wait_gather(b)
            @pl.when(c + NBUF < NCHUNK)
            def _():
                issue_idx(c + NBUF, b)
            @pl.when(c + 1 < NCHUNK)
            def _():
                wait_idx(c + 1, (b + 1) % NBUF)
                issue_gather((b + 1) % NBUF)
            @pl.when(c >= NBUF)
            def _():
                wait_out(c - NBUF, b)
            for r in range(CR):
                for k in range(D // L):
                    acc = rows_v[b, r * NC, pl.ds(k * L, L)]
                    for j in range(1, NC):
                        acc = acc + rows_v[b, r * NC + j, pl.ds(k * L, L)]
                    out_v[b, r, pl.ds(k * L, L)] = acc
                out_v[b, r, pl.ds(D, DC)] = cont_v[c * CR + r, :]
            issue_out(c, b)
        return carry

    lax.fori_loop(0, NCHUNK // NBUF, body, 0)
    for b in range(NBUF):
        wait_out(NCHUNK - NBUF + b, b)


def kernel(ContTensor, CatTensor, LabelTensor, DoseTensor, TimeDiffTensor,
           VTensor, VancoElTensor, PtList, LengList, embed_weight):
    idx_flat = CatTensor.astype(jnp.int32).reshape(ROWS * NC)
    cont_flat = ContTensor.reshape(ROWS, DC)
    out = _emb_kernel(idx_flat, cont_flat, embed_weight)
    outEmb = out.reshape(B, T, DOUT)
    return (outEmb, LabelTensor, LengList, DoseTensor, TimeDiffTensor,
            VTensor, VancoElTensor, PtList)


# speedup vs baseline: 2.1184x; 1.1785x over previous
import functools

import jax
import jax.numpy as jnp
from jax import lax
from jax.experimental import pallas as pl
from jax.experimental.pallas import tpu as pltpu
from jax.experimental.pallas import tpu_sc as plsc

B, T, NC, DC = 1024, 50, 26, 16
V, D = 1000000, 64
DOUT = D + DC
L = 16

_NUM_CORES = 2
_NUM_SUBCORES = 16
NW = _NUM_CORES * _NUM_SUBCORES
BB = B // NW

_mesh = plsc.VectorSubcoreMesh(core_axis_name="c", subcore_axis_name="s")


@functools.partial(
    pl.kernel,
    mesh=_mesh,
    out_type=jax.ShapeDtypeStruct((B, T, DOUT), jnp.float32),
    compiler_params=pltpu.CompilerParams(use_tc_tiling_on_sc=False),
    scratch_types=[
        pltpu.VMEM((2, NC, BB), jnp.int32),
        pltpu.VMEM((2, NC, BB, D), jnp.float32),
        pltpu.VMEM((2, BB, DC), jnp.float32),
        pltpu.VMEM((2, BB, DOUT), jnp.float32),
        pltpu.SemaphoreType.DMA((2,)),
        pltpu.SemaphoreType.DMA((2,)),
        pltpu.SemaphoreType.DMA((2,)),
        pltpu.SemaphoreType.DMA((2,)),
    ],
)
def _emb_kernel(idx_hbm, cont_hbm, table_hbm, out_hbm,
                idx_v, rows_v, cont_v, out_v,
                idx_sem, cont_sem, gather_sem, out_sem):
    wid = lax.axis_index("s") * _NUM_CORES + lax.axis_index("c")
    b0 = wid * BB

    def issue_idx(t, p):
        pltpu.async_copy(idx_hbm.at[:, t, pl.ds(b0, BB)], idx_v.at[p],
                         idx_sem.at[p])

    def wait_idx(t, p):
        pltpu.make_async_copy(idx_hbm.at[:, t, pl.ds(b0, BB)], idx_v.at[p],
                              idx_sem.at[p]).wait()

    def issue_cont(t, p):
        pltpu.async_copy(cont_hbm.at[pl.ds(b0, BB), t, :], cont_v.at[p],
                         cont_sem.at[p])

    def wait_cont(t, p):
        pltpu.make_async_copy(cont_hbm.at[pl.ds(b0, BB), t, :], cont_v.at[p],
                              cont_sem.at[p]).wait()

    def issue_gathers(p):
        for c in range(NC):
            pltpu.async_copy(table_hbm.at[idx_v.at[p].at[c]],
                             rows_v.at[p].at[c], gather_sem.at[p])

    def wait_gathers(p):
        for c in range(NC):
            pltpu.make_async_copy(table_hbm.at[idx_v.at[p].at[c]],
                                  rows_v.at[p].at[c], gather_sem.at[p]).wait()

    def issue_out(t, p):
        pltpu.async_copy(out_v.at[p], out_hbm.at[pl.ds(b0, BB), t, :],
                         out_sem.at[p])

    def wait_out(t, p):
        pltpu.make_async_copy(out_v.at[p], out_hbm.at[pl.ds(b0, BB), t, :],
                              out_sem.at[p]).wait()

    issue_idx(0, 0)
    issue_cont(0, 0)
    wait_idx(0, 0)
    issue_gathers(0)
    issue_idx(1, 1)
    issue_cont(1, 1)

    def body(ii, carry):
        for p in range(2):
            t = ii * 2 + p
            wait_gathers(p)
            @pl.when(t + 1 < T)
            def _():
                wait_idx(t + 1, 1 - p)
                issue_gathers(1 - p)
            @pl.when(t + 2 < T)
            def _():
                issue_idx(t + 2, p)
            wait_cont(t, p)
            @pl.when(t >= 2)
            def _():
                wait_out(t - 2, p)

            def reduce_one(b, carry2):
                for k in range(D // L):
                    acc = rows_v[p, 0, b, pl.ds(k * L, L)]
                    for c in range(1, NC):
                        acc = acc + rows_v[p, c, b, pl.ds(k * L, L)]
                    out_v[p, b, pl.ds(k * L, L)] = acc
                out_v[p, b, pl.ds(D, DC)] = cont_v[p, b, :]
                return carry2

            lax.fori_loop(0, BB, reduce_one, 0)
            @pl.when(t + 2 < T)
            def _():
                issue_cont(t + 2, p)
            issue_out(t, p)
        return carry

    lax.fori_loop(0, T // 2, body, 0)
    wait_out(T - 2, 0)
    wait_out(T - 1, 1)


def kernel(ContTensor, CatTensor, LabelTensor, DoseTensor, TimeDiffTensor,
           VTensor, VancoElTensor, PtList, LengList, embed_weight):
    idx_t = jnp.transpose(CatTensor.astype(jnp.int32), (2, 1, 0))
    out = _emb_kernel(idx_t, ContTensor, embed_weight)
    return (out, LabelTensor, LengList, DoseTensor, TimeDiffTensor,
            VTensor, VancoElTensor, PtList)
